# Initial kernel scaffold; baseline (speedup 1.0000x reference)
import jax
import jax.numpy as jnp
from jax.experimental import pallas as pl


def kernel(entities, ent_embs):
    raise NotImplementedError("write your pallas kernel here")


# speedup vs baseline: 1.6851x; 1.6851x over previous
import functools

import jax
import jax.numpy as jnp
from jax import lax
from jax.experimental import pallas as pl
from jax.experimental.pallas import tpu as pltpu
from jax.experimental.pallas import tpu_sc as plsc

DIM = 64
K = 128


@functools.lru_cache(maxsize=None)
def _make_lookup(n_idx: int, n_ent: int, dim: int):
    info = plsc.get_sparse_core_info()
    nw = info.num_cores * info.num_subcores
    n_chunks = n_idx // K
    chunks_per_w = n_chunks // nw

    mesh = plsc.VectorSubcoreMesh(core_axis_name="c", subcore_axis_name="s")

    @functools.partial(
        pl.kernel,
        mesh=mesh,
        out_type=jax.ShapeDtypeStruct((n_chunks, K, dim), jnp.float32),
        scratch_types=[
            pltpu.VMEM((chunks_per_w, K), jnp.int32),
            pltpu.VMEM((K, dim), jnp.float32),
            pltpu.SemaphoreType.DMA,
        ],
        compiler_params=pltpu.CompilerParams(use_tc_tiling_on_sc=False),
    )
    def lookup(idx_hbm, table_hbm, out_hbm, idx_v, rows_v, sem):
        wid = lax.axis_index("s") * info.num_cores + lax.axis_index("c")
        row0 = wid * chunks_per_w
        pltpu.sync_copy(idx_hbm.at[pl.ds(row0, chunks_per_w)], idx_v)

        def body(j, carry):
            pltpu.async_copy(table_hbm.at[idx_v.at[j]], rows_v, sem).wait()
            pltpu.sync_copy(rows_v, out_hbm.at[row0 + j])
            return carry

        lax.fori_loop(0, chunks_per_w, body, 0)

    return lookup


@jax.jit
def kernel(entities, ent_embs):
    batch, hist = entities.shape
    n_ent, dim = ent_embs.shape
    idx = entities.reshape(-1).astype(jnp.int32)
    n_idx = idx.shape[0]
    idx2d = idx.reshape(n_idx // K, K)
    out = _make_lookup(n_idx, n_ent, dim)(idx2d, ent_embs)
    return out.reshape(batch, hist, dim)


# speedup vs baseline: 1.8274x; 1.0844x over previous
import functools

import jax
import jax.numpy as jnp
from jax import lax
from jax.experimental import pallas as pl
from jax.experimental.pallas import tpu as pltpu
from jax.experimental.pallas import tpu_sc as plsc

DIM = 64
K = 128
CB = 4


@functools.lru_cache(maxsize=None)
def _make_lookup(n_idx: int, n_ent: int, dim: int):
    info = plsc.get_sparse_core_info()
    nw = info.num_cores * info.num_subcores
    n_chunks = n_idx // K
    chunks_per_w = n_chunks // nw
    steps_per_w = chunks_per_w // CB

    mesh = plsc.VectorSubcoreMesh(core_axis_name="c", subcore_axis_name="s")

    @functools.partial(
        pl.kernel,
        mesh=mesh,
        out_type=jax.ShapeDtypeStruct((n_idx, dim), jnp.float32),
        scratch_types=[
            pltpu.VMEM((steps_per_w * CB * K,), jnp.int32),
            pltpu.VMEM((CB * K, dim), jnp.float32),
            pltpu.SemaphoreType.DMA,
        ],
        compiler_params=pltpu.CompilerParams(use_tc_tiling_on_sc=False),
    )
    def lookup(idx_hbm, table_hbm, out_hbm, idx_v, rows_v, sem):
        wid = lax.axis_index("s") * info.num_cores + lax.axis_index("c")
        step0 = wid * steps_per_w
        pltpu.sync_copy(
            idx_hbm.at[pl.ds(step0 * CB * K, steps_per_w * CB * K)], idx_v)

        def body(j, carry):
            pltpu.async_copy(
                table_hbm.at[idx_v.at[pl.ds(j * CB * K, CB * K)]],
                rows_v, sem).wait()
            pltpu.sync_copy(
                rows_v, out_hbm.at[pl.ds((step0 + j) * CB * K, CB * K)])
            return carry

        lax.fori_loop(0, steps_per_w, body, 0)

    return lookup


@jax.jit
def kernel(entities, ent_embs):
    batch, hist = entities.shape
    n_ent, dim = ent_embs.shape
    idx = entities.reshape(-1).astype(jnp.int32)
    n_idx = idx.shape[0]
    out = _make_lookup(n_idx, n_ent, dim)(idx, ent_embs)
    return out.reshape(batch, hist, dim)


# speedup vs baseline: 1.8821x; 1.0300x over previous
import functools

import jax
import jax.numpy as jnp
from jax import lax
from jax.experimental import pallas as pl
from jax.experimental.pallas import tpu as pltpu
from jax.experimental.pallas import tpu_sc as plsc

DIM = 64
CHUNK = 256
NBUF = 4


@functools.lru_cache(maxsize=None)
def _make_lookup(n_idx: int, n_ent: int, dim: int):
    info = plsc.get_sparse_core_info()
    nw = info.num_cores * info.num_subcores
    idx_per_w = n_idx // nw
    steps_per_w = idx_per_w // CHUNK
    n_outer = steps_per_w // NBUF
    assert steps_per_w % NBUF == 0 and n_idx % (nw * CHUNK) == 0

    mesh = plsc.VectorSubcoreMesh(core_axis_name="c", subcore_axis_name="s")

    @functools.partial(
        pl.kernel,
        mesh=mesh,
        out_type=jax.ShapeDtypeStruct((n_idx, dim), jnp.float32),
        scratch_types=[
            pltpu.VMEM((idx_per_w,), jnp.int32),
            pltpu.VMEM((NBUF, CHUNK, dim), jnp.float32),
            pltpu.SemaphoreType.DMA((NBUF,)),
            pltpu.SemaphoreType.DMA((NBUF,)),
        ],
        compiler_params=pltpu.CompilerParams(use_tc_tiling_on_sc=False),
    )
    def lookup(idx_hbm, table_hbm, out_hbm, idx_v, rows_v, sem_g, sem_w):
        wid = lax.axis_index("s") * info.num_cores + lax.axis_index("c")
        base = wid * idx_per_w
        pltpu.sync_copy(idx_hbm.at[pl.ds(base, idx_per_w)], idx_v)

        def outer(t, carry):
            gathers = []
            for b in range(NBUF):
                j = t * NBUF + b

                @pl.when(t > 0)
                def _wait_write():
                    pltpu.make_async_copy(
                        rows_v.at[b], out_hbm.at[pl.ds(0, CHUNK)], sem_w.at[b]
                    ).wait()

                gathers.append(
                    pltpu.async_copy(
                        table_hbm.at[idx_v.at[pl.ds(j * CHUNK, CHUNK)]],
                        rows_v.at[b],
                        sem_g.at[b],
                    )
                )
            for b in range(NBUF):
                j = t * NBUF + b
                gathers[b].wait()
                pltpu.async_copy(
                    rows_v.at[b],
                    out_hbm.at[pl.ds(base + j * CHUNK, CHUNK)],
                    sem_w.at[b],
                )
            return carry

        lax.fori_loop(0, n_outer, outer, 0)
        for b in range(NBUF):
            pltpu.make_async_copy(
                rows_v.at[b], out_hbm.at[pl.ds(0, CHUNK)], sem_w.at[b]
            ).wait()

    return lookup


@jax.jit
def kernel(entities, ent_embs):
    batch, hist = entities.shape
    n_ent, dim = ent_embs.shape
    idx = entities.reshape(-1).astype(jnp.int32)
    n_idx = idx.shape[0]
    out = _make_lookup(n_idx, n_ent, dim)(idx, ent_embs)
    return out.reshape(batch, hist, dim)
